# baseline (speedup 1.0000x reference)
import jax
import jax.numpy as jnp
from jax.experimental import pallas as pl
from jax.experimental.pallas import tpu as pltpu

N_DIMS = 3
ATOM_NF = 16
RESIDUE_NF = 21
JOINT_NF = 16
HIDDEN_NF = 64
N_LAYERS = 4
NORM_FACTOR = 100.0
N_BATCH = 16
N_ATOMS = 320
N_RES = 1600
N_NODES = N_ATOMS + N_RES

TJ = 128
TI = 32
TJH = TJ // 2
FOLD = 2 * HIDDEN_NF
N2 = N_NODES + N_BATCH * (TJ - 1) + 32
N2 = ((N2 + TJ - 1) // TJ) * TJ
NT2 = N2 // TJ


def _silu(x):
    return 0.5 * x * (1.0 + jnp.tanh(0.5 * x))


def _silu_fast(x):
    t = 0.5 * x
    th = jnp.tanh(t.astype(jnp.bfloat16)).astype(jnp.float32)
    return t + t * th


def _pre_kernel(xa_ref, xr_ref, t_ref,
                wa1, ba1, wa2, ba2,
                wr1, br1, wr2, br2,
                we, be,
                out_ref):
    xa = xa_ref[...]
    xr = xr_ref[...]
    ha = xa[:, N_DIMS:]
    hr = xr[:, N_DIMS:]
    ha = _silu(ha @ wa1[...] + ba1[...]) @ wa2[...] + ba2[...]
    hr = _silu(hr @ wr1[...] + br1[...]) @ wr2[...] + br2[...]
    we_full = we[...]
    we_x = we_full[:N_DIMS, :]
    we_h = we_full[N_DIMS:N_DIMS + JOINT_NF, :]
    we_t = we_full[N_DIMS + JOINT_NF:, :]
    tval = t_ref[...]
    tcontrib = tval * we_t
    fa = xa[:, :N_DIMS] @ we_x + ha @ we_h + tcontrib + be[...]
    fr = xr[:, :N_DIMS] @ we_x + hr @ we_h + tcontrib + be[...]
    out_ref[:N_ATOMS, :] = fa
    out_ref[N_ATOMS:N_NODES, :] = fr
    out_ref[N_NODES:, :] = jnp.zeros((8, HIDDEN_NF), jnp.float32)


def _copy_jobs(jobs_ref, src_full, dst_ref, n_jobs):
    def job(b, _):
        src = jobs_ref[0, b]
        dstt = jobs_ref[1, b]
        ln = jobs_ref[2, b]

        def chunk(k, __):
            st = jnp.maximum(0, jnp.minimum(k * 8, ln - 8))
            dst_ref[pl.ds(dstt + st, 8), :] = src_full[pl.ds(src + st, 8), :]
            return 0

        return jax.lax.fori_loop(0, (ln + 7) // 8, chunk, 0)

    jax.lax.fori_loop(0, n_jobs, job, 0)


def _pack_kernel(jobs_ref, feat_ref, out_ref):
    out_ref[...] = jnp.zeros((N2, HIDDEN_NF), jnp.float32)
    _copy_jobs(jobs_ref, feat_ref, out_ref, 2 * N_BATCH)


def _unpack_kernel(jobs_ref, featpk_ref, out_ref):
    out_ref[...] = jnp.zeros((N_NODES + 8, HIDDEN_NF), jnp.float32)
    _copy_jobs(jobs_ref, featpk_ref, out_ref, 2 * N_BATCH)


def _gcl_kernel(dst_ref,
                feat0_ref,
                bias2_ref,
                w1a, b1, w1b, w2d,
                w3f, w3a, b3, w4, b4,
                out_ref,
                fbuf,
                a2buf,
                bfbuf,
                aggbuf):
    l = pl.program_id(0)
    cur = jax.lax.rem(l, 2)
    nxt = 1 - cur

    @pl.when(l == 0)
    def _copy_in():
        fbuf[0] = feat0_ref[...]

    f = fbuf[cur]
    a = f @ w1a[0] + b1[0]
    a2buf[...] = jnp.concatenate([a, a], axis=1)
    b = f @ w1b[0]
    br = b.reshape(NT2, 2, TJH, HIDDEN_NF)
    bfbuf[...] = jnp.concatenate([br[:, 0], br[:, 1]], axis=-1)
    aggbuf[...] = jnp.zeros((N2, HIDDEN_NF), jnp.float32)

    def batch_loop(bi, _):
        c0 = dst_ref[0, bi] // TJ
        c1 = dst_ref[0, bi + 1] // TJ
        r0 = dst_ref[0, bi] // TI
        r1 = dst_ref[0, bi + 1] // TI

        def row(rt, __):
            ioff = rt * TI
            a2 = a2buf[pl.ds(ioff, TI), :]

            def col(j, acc):
                bf = bfbuf[j]
                u = _silu(a2[:, None, :] + bf[None, :, :])
                mp = u.reshape(TI * TJH, FOLD) @ w2d[0]
                m = _silu(mp.reshape(TI, TJH, FOLD) + bias2_ref[0, j])
                s = jnp.sum(m, axis=1)
                return acc + s[:, :HIDDEN_NF] + s[:, HIDDEN_NF:]

            acc = jax.lax.fori_loop(c0, c1, col,
                                    jnp.zeros((TI, HIDDEN_NF), jnp.float32))
            aggbuf[pl.ds(ioff, TI), :] = acc
            return 0

        jax.lax.fori_loop(r0, r1, row, 0)
        return 0

    jax.lax.fori_loop(0, N_BATCH, batch_loop, 0)

    agg = aggbuf[...] * (1.0 / NORM_FACTOR)
    tmp = _silu(f @ w3f[0] + agg @ w3a[0] + b3[0])
    newf = f + tmp @ w4[0] + b4[0]
    fbuf[nxt] = newf

    @pl.when(l == N_LAYERS - 1)
    def _emit():
        out_ref[...] = newf


def _post_kernel(feat_ref, maski_ref,
                 weo, beo,
                 wad1, bad1, wad2, bad2,
                 wrd1, brd1, wrd2, brd2,
                 outa_ref, outr_ref):
    feat = feat_ref[...]
    out = feat @ weo[...] + beo[...]
    vel = out[:, :N_DIMS]
    hfin = out[:, N_DIMS:N_DIMS + JOINT_NF]
    ha = _silu(hfin[:N_ATOMS] @ wad1[...] + bad1[...]) @ wad2[...] + bad2[...]
    hr = _silu(hfin[N_ATOMS:] @ wrd1[...] + brd1[...]) @ wrd2[...] + brd2[...]
    maski = maski_ref[...]
    batches = jax.lax.broadcasted_iota(jnp.int32, (N_NODES, N_BATCH), 1)
    onehot = (maski == batches).astype(jnp.float32)
    seg = jax.lax.dot_general(onehot, vel, (((0,), (0,)), ((), ())))
    cnt = jnp.sum(onehot, axis=0, keepdims=True).T
    mean = seg / jnp.maximum(cnt, 1.0)
    vel = vel - onehot @ mean
    outa_ref[...] = jnp.concatenate([vel[:N_ATOMS], ha], axis=1)
    outr_ref[...] = jnp.concatenate([vel[N_ATOMS:], hr], axis=1)


@jax.jit
def _run(xh_atoms, xh_residues, t, mask_atoms, mask_residues, params):
    mask = jnp.concatenate([mask_atoms, mask_residues]).astype(jnp.int32)
    batch_ids = jnp.arange(N_BATCH + 1, dtype=jnp.int32)
    apos = jnp.searchsorted(mask_atoms.astype(jnp.int32), batch_ids).astype(jnp.int32)
    rpos = jnp.searchsorted(mask_residues.astype(jnp.int32), batch_ids).astype(jnp.int32)
    ca = apos[1:] - apos[:-1]
    cr = rpos[1:] - rpos[:-1]

    cnt = ca + cr
    padc = ((cnt + TJ - 1) // TJ) * TJ
    dst = jnp.concatenate([jnp.zeros((1,), jnp.int32),
                           jnp.cumsum(padc).astype(jnp.int32)])
    npad = dst[-1]

    rowidx = jnp.arange(N2, dtype=jnp.int32)
    brow = jnp.clip(jnp.searchsorted(dst, rowidx, side="right") - 1, 0, N_BATCH - 1)
    off = rowidx - dst[brow]
    valid = (off < cnt[brow]) & (rowidx < npad)

    src_a, dst_a = apos[:-1], dst[:-1]
    src_r, dst_r = N_ATOMS + rpos[:-1], dst[:-1] + ca
    jobs_pack = jnp.stack([
        jnp.stack([src_a, src_r], axis=1).reshape(-1),
        jnp.stack([dst_a, dst_r], axis=1).reshape(-1),
        jnp.stack([ca, cr], axis=1).reshape(-1),
    ]).astype(jnp.int32)
    jobs_unpack = jnp.stack([
        jnp.concatenate([dst_a, dst_r]),
        jnp.concatenate([src_a, src_r]),
        jnp.concatenate([ca, cr]),
    ]).astype(jnp.int32)

    vf = jnp.where(valid, 0.0, -1e9).astype(jnp.float32).reshape(NT2, 2, TJH)
    vbias = jnp.concatenate(
        [jnp.broadcast_to(vf[:, 0, :, None], (NT2, TJH, HIDDEN_NF)),
         jnp.broadcast_to(vf[:, 1, :, None], (NT2, TJH, HIDDEN_NF))], axis=2)

    p = params

    def lin(name, sub=None):
        q = p[name] if sub is None else p[name][sub]
        return q["w"], q["b"].reshape(1, -1)

    wa1, ba1 = lin("atom_encoder", "l1")
    wa2, ba2 = lin("atom_encoder", "l2")
    wr1, br1 = lin("residue_encoder", "l1")
    wr2, br2 = lin("residue_encoder", "l2")
    we, be = p["gnn"]["embedding"]["w"], p["gnn"]["embedding"]["b"].reshape(1, -1)

    feat = pl.pallas_call(
        _pre_kernel,
        out_shape=jax.ShapeDtypeStruct((N_NODES + 8, HIDDEN_NF), jnp.float32),
    )(xh_atoms, xh_residues, t.reshape(1, 1),
      wa1, ba1, wa2, ba2, wr1, br1, wr2, br2, we, be)

    featpk = pl.pallas_call(
        _pack_kernel,
        in_specs=[
            pl.BlockSpec(memory_space=pltpu.SMEM),
            pl.BlockSpec((N_NODES + 8, HIDDEN_NF), lambda: (0, 0)),
        ],
        out_specs=pl.BlockSpec((N2, HIDDEN_NF), lambda: (0, 0)),
        out_shape=jax.ShapeDtypeStruct((N2, HIDDEN_NF), jnp.float32),
    )(jobs_pack, feat)

    gcl = p["gnn"]["gcl"]

    def stack(path):
        return jnp.stack([path(layer) for layer in gcl])

    def bdiag(w):
        return jnp.kron(jnp.eye(2, dtype=w.dtype), w)

    w1a_s = stack(lambda q: q["edge_mlp"]["l1"]["w"][:HIDDEN_NF, :])
    w1b_s = stack(lambda q: q["edge_mlp"]["l1"]["w"][HIDDEN_NF:, :])
    b1_s = stack(lambda q: q["edge_mlp"]["l1"]["b"].reshape(1, HIDDEN_NF))
    w2d_s = stack(lambda q: bdiag(q["edge_mlp"]["l2"]["w"]))
    b2d_s = stack(lambda q: jnp.tile(q["edge_mlp"]["l2"]["b"].reshape(1, HIDDEN_NF), (1, 2)))
    w3f_s = stack(lambda q: q["node_mlp"]["l1"]["w"][:HIDDEN_NF, :])
    w3a_s = stack(lambda q: q["node_mlp"]["l1"]["w"][HIDDEN_NF:, :])
    b3_s = stack(lambda q: q["node_mlp"]["l1"]["b"].reshape(1, HIDDEN_NF))
    w4_s = stack(lambda q: q["node_mlp"]["l2"]["w"])
    b4_s = stack(lambda q: q["node_mlp"]["l2"]["b"].reshape(1, HIDDEN_NF))

    bias2_s = b2d_s[:, None, :, :] + vbias[None, :, :, :]

    def wspec(shape):
        nd = len(shape) - 1
        return pl.BlockSpec((1,) + shape[1:], lambda l: (l,) + (0,) * nd)

    featpk = pl.pallas_call(
        _gcl_kernel,
        grid=(N_LAYERS,),
        in_specs=[
            pl.BlockSpec(memory_space=pltpu.SMEM),
            pl.BlockSpec((N2, HIDDEN_NF), lambda l: (0, 0)),
            wspec(bias2_s.shape),
            wspec(w1a_s.shape), wspec(b1_s.shape), wspec(w1b_s.shape),
            wspec(w2d_s.shape),
            wspec(w3f_s.shape), wspec(w3a_s.shape), wspec(b3_s.shape),
            wspec(w4_s.shape), wspec(b4_s.shape),
        ],
        out_specs=pl.BlockSpec((N2, HIDDEN_NF), lambda l: (0, 0)),
        out_shape=jax.ShapeDtypeStruct((N2, HIDDEN_NF), jnp.float32),
        scratch_shapes=[pltpu.VMEM((2, N2, HIDDEN_NF), jnp.float32),
                        pltpu.VMEM((N2, FOLD), jnp.float32),
                        pltpu.VMEM((NT2, TJH, FOLD), jnp.float32),
                        pltpu.VMEM((N2, HIDDEN_NF), jnp.float32)],
    )(dst.reshape(1, N_BATCH + 1), featpk, bias2_s,
      w1a_s, b1_s, w1b_s, w2d_s, w3f_s, w3a_s, b3_s, w4_s, b4_s)

    featu = pl.pallas_call(
        _unpack_kernel,
        in_specs=[
            pl.BlockSpec(memory_space=pltpu.SMEM),
            pl.BlockSpec((N2, HIDDEN_NF), lambda: (0, 0)),
        ],
        out_specs=pl.BlockSpec((N_NODES + 8, HIDDEN_NF), lambda: (0, 0)),
        out_shape=jax.ShapeDtypeStruct((N_NODES + 8, HIDDEN_NF), jnp.float32),
    )(jobs_unpack, featpk)[:N_NODES]

    weo, beo = p["gnn"]["embedding_out"]["w"], p["gnn"]["embedding_out"]["b"].reshape(1, -1)
    wad1, bad1 = lin("atom_decoder", "l1")
    wad2, bad2 = lin("atom_decoder", "l2")
    wrd1, brd1 = lin("residue_decoder", "l1")
    wrd2, brd2 = lin("residue_decoder", "l2")

    outa, outr = pl.pallas_call(
        _post_kernel,
        out_shape=(jax.ShapeDtypeStruct((N_ATOMS, N_DIMS + ATOM_NF), jnp.float32),
                   jax.ShapeDtypeStruct((N_RES, N_DIMS + RESIDUE_NF), jnp.float32)),
    )(featu, mask.reshape(N_NODES, 1),
      weo, beo, wad1, bad1, wad2, bad2, wrd1, brd1, wrd2, brd2)

    return outa, outr


def kernel(xh_atoms, xh_residues, xh_intersh, xh_intershp, t,
           mask_atoms, mask_residues, mask_intersh, mask_intershp, params):
    return _run(xh_atoms, xh_residues, t, mask_atoms, mask_residues, params)


# speedup vs baseline: 1.0537x; 1.0537x over previous
import jax
import jax.numpy as jnp
from jax.experimental import pallas as pl
from jax.experimental.pallas import tpu as pltpu

N_DIMS = 3
ATOM_NF = 16
RESIDUE_NF = 21
JOINT_NF = 16
HIDDEN_NF = 64
N_LAYERS = 4
NORM_FACTOR = 100.0
N_BATCH = 16
N_ATOMS = 320
N_RES = 1600
N_NODES = N_ATOMS + N_RES

TJ = 128
TI = 64
TJH = TJ // 2
FOLD = 2 * HIDDEN_NF
N2 = N_NODES + N_BATCH * (TJ - 1) + 32
N2 = ((N2 + TJ - 1) // TJ) * TJ
NT2 = N2 // TJ


def _silu(x):
    return 0.5 * x * (1.0 + jnp.tanh(0.5 * x))


def _silu_fast(x):
    t = 0.5 * x
    th = jnp.tanh(t.astype(jnp.bfloat16)).astype(jnp.float32)
    return t + t * th


def _pre_kernel(xa_ref, xr_ref, t_ref,
                wa1, ba1, wa2, ba2,
                wr1, br1, wr2, br2,
                we, be,
                out_ref):
    xa = xa_ref[...]
    xr = xr_ref[...]
    ha = xa[:, N_DIMS:]
    hr = xr[:, N_DIMS:]
    ha = _silu(ha @ wa1[...] + ba1[...]) @ wa2[...] + ba2[...]
    hr = _silu(hr @ wr1[...] + br1[...]) @ wr2[...] + br2[...]
    we_full = we[...]
    we_x = we_full[:N_DIMS, :]
    we_h = we_full[N_DIMS:N_DIMS + JOINT_NF, :]
    we_t = we_full[N_DIMS + JOINT_NF:, :]
    tval = t_ref[...]
    tcontrib = tval * we_t
    fa = xa[:, :N_DIMS] @ we_x + ha @ we_h + tcontrib + be[...]
    fr = xr[:, :N_DIMS] @ we_x + hr @ we_h + tcontrib + be[...]
    out_ref[:N_ATOMS, :] = fa
    out_ref[N_ATOMS:N_NODES, :] = fr
    out_ref[N_NODES:, :] = jnp.zeros((8, HIDDEN_NF), jnp.float32)


def _copy_jobs(jobs_ref, src_full, dst_ref, n_jobs):
    def job(b, _):
        src = jobs_ref[0, b]
        dstt = jobs_ref[1, b]
        ln = jobs_ref[2, b]

        def chunk(k, __):
            st = jnp.maximum(0, jnp.minimum(k * 8, ln - 8))
            dst_ref[pl.ds(dstt + st, 8), :] = src_full[pl.ds(src + st, 8), :]
            return 0

        return jax.lax.fori_loop(0, (ln + 7) // 8, chunk, 0)

    jax.lax.fori_loop(0, n_jobs, job, 0)


def _pack_kernel(jobs_ref, feat_ref, out_ref):
    out_ref[...] = jnp.zeros((N2, HIDDEN_NF), jnp.float32)
    _copy_jobs(jobs_ref, feat_ref, out_ref, 2 * N_BATCH)


def _unpack_kernel(jobs_ref, featpk_ref, out_ref):
    out_ref[...] = jnp.zeros((N_NODES + 8, HIDDEN_NF), jnp.float32)
    _copy_jobs(jobs_ref, featpk_ref, out_ref, 2 * N_BATCH)


def _gcl_kernel(dst_ref,
                feat0_ref,
                bias2_ref,
                w1a, b1, w1b, w2d,
                w3f, w3a, b3, w4, b4,
                out_ref,
                fbuf,
                a2buf,
                bfbuf,
                aggbuf):
    l = pl.program_id(0)
    cur = jax.lax.rem(l, 2)
    nxt = 1 - cur

    @pl.when(l == 0)
    def _copy_in():
        fbuf[0] = feat0_ref[...]

    f = fbuf[cur]
    a = f @ w1a[0] + b1[0]
    a2buf[...] = jnp.concatenate([a, a], axis=1)
    b = f @ w1b[0]
    br = b.reshape(NT2, 2, TJH, HIDDEN_NF)
    bfbuf[...] = jnp.concatenate([br[:, 0], br[:, 1]], axis=-1)
    aggbuf[...] = jnp.zeros((N2, HIDDEN_NF), jnp.float32)

    def batch_loop(bi, _):
        c0 = dst_ref[0, bi] // TJ
        c1 = dst_ref[0, bi + 1] // TJ
        r0 = dst_ref[0, bi] // TI
        r1 = dst_ref[0, bi + 1] // TI

        def row(rt, __):
            ioff = rt * TI
            a2 = a2buf[pl.ds(ioff, TI), :]

            def col(j, acc):
                bf = bfbuf[j]
                u = _silu(a2[:, None, :] + bf[None, :, :])
                mp = u.reshape(TI * TJH, FOLD) @ w2d[0]
                m = _silu(mp.reshape(TI, TJH, FOLD) + bias2_ref[0, j])
                s = jnp.sum(m, axis=1)
                return acc + s[:, :HIDDEN_NF] + s[:, HIDDEN_NF:]

            acc = jax.lax.fori_loop(c0, c1, col,
                                    jnp.zeros((TI, HIDDEN_NF), jnp.float32))
            aggbuf[pl.ds(ioff, TI), :] = acc
            return 0

        jax.lax.fori_loop(r0, r1, row, 0)
        return 0

    jax.lax.fori_loop(0, N_BATCH, batch_loop, 0)

    agg = aggbuf[...] * (1.0 / NORM_FACTOR)
    tmp = _silu(f @ w3f[0] + agg @ w3a[0] + b3[0])
    newf = f + tmp @ w4[0] + b4[0]
    fbuf[nxt] = newf

    @pl.when(l == N_LAYERS - 1)
    def _emit():
        out_ref[...] = newf


def _post_kernel(feat_ref, maski_ref,
                 weo, beo,
                 wad1, bad1, wad2, bad2,
                 wrd1, brd1, wrd2, brd2,
                 outa_ref, outr_ref):
    feat = feat_ref[...]
    out = feat @ weo[...] + beo[...]
    vel = out[:, :N_DIMS]
    hfin = out[:, N_DIMS:N_DIMS + JOINT_NF]
    ha = _silu(hfin[:N_ATOMS] @ wad1[...] + bad1[...]) @ wad2[...] + bad2[...]
    hr = _silu(hfin[N_ATOMS:] @ wrd1[...] + brd1[...]) @ wrd2[...] + brd2[...]
    maski = maski_ref[...]
    batches = jax.lax.broadcasted_iota(jnp.int32, (N_NODES, N_BATCH), 1)
    onehot = (maski == batches).astype(jnp.float32)
    seg = jax.lax.dot_general(onehot, vel, (((0,), (0,)), ((), ())))
    cnt = jnp.sum(onehot, axis=0, keepdims=True).T
    mean = seg / jnp.maximum(cnt, 1.0)
    vel = vel - onehot @ mean
    outa_ref[...] = jnp.concatenate([vel[:N_ATOMS], ha], axis=1)
    outr_ref[...] = jnp.concatenate([vel[N_ATOMS:], hr], axis=1)


@jax.jit
def _run(xh_atoms, xh_residues, t, mask_atoms, mask_residues, params):
    mask = jnp.concatenate([mask_atoms, mask_residues]).astype(jnp.int32)
    batch_ids = jnp.arange(N_BATCH + 1, dtype=jnp.int32)
    apos = jnp.searchsorted(mask_atoms.astype(jnp.int32), batch_ids).astype(jnp.int32)
    rpos = jnp.searchsorted(mask_residues.astype(jnp.int32), batch_ids).astype(jnp.int32)
    ca = apos[1:] - apos[:-1]
    cr = rpos[1:] - rpos[:-1]

    cnt = ca + cr
    padc = ((cnt + TJ - 1) // TJ) * TJ
    dst = jnp.concatenate([jnp.zeros((1,), jnp.int32),
                           jnp.cumsum(padc).astype(jnp.int32)])
    npad = dst[-1]

    rowidx = jnp.arange(N2, dtype=jnp.int32)
    brow = jnp.clip(jnp.searchsorted(dst, rowidx, side="right") - 1, 0, N_BATCH - 1)
    off = rowidx - dst[brow]
    valid = (off < cnt[brow]) & (rowidx < npad)

    src_a, dst_a = apos[:-1], dst[:-1]
    src_r, dst_r = N_ATOMS + rpos[:-1], dst[:-1] + ca
    jobs_pack = jnp.stack([
        jnp.stack([src_a, src_r], axis=1).reshape(-1),
        jnp.stack([dst_a, dst_r], axis=1).reshape(-1),
        jnp.stack([ca, cr], axis=1).reshape(-1),
    ]).astype(jnp.int32)
    jobs_unpack = jnp.stack([
        jnp.concatenate([dst_a, dst_r]),
        jnp.concatenate([src_a, src_r]),
        jnp.concatenate([ca, cr]),
    ]).astype(jnp.int32)

    vf = jnp.where(valid, 0.0, -1e9).astype(jnp.float32).reshape(NT2, 2, TJH)
    vbias = jnp.concatenate(
        [jnp.broadcast_to(vf[:, 0, :, None], (NT2, TJH, HIDDEN_NF)),
         jnp.broadcast_to(vf[:, 1, :, None], (NT2, TJH, HIDDEN_NF))], axis=2)

    p = params

    def lin(name, sub=None):
        q = p[name] if sub is None else p[name][sub]
        return q["w"], q["b"].reshape(1, -1)

    wa1, ba1 = lin("atom_encoder", "l1")
    wa2, ba2 = lin("atom_encoder", "l2")
    wr1, br1 = lin("residue_encoder", "l1")
    wr2, br2 = lin("residue_encoder", "l2")
    we, be = p["gnn"]["embedding"]["w"], p["gnn"]["embedding"]["b"].reshape(1, -1)

    feat = pl.pallas_call(
        _pre_kernel,
        out_shape=jax.ShapeDtypeStruct((N_NODES + 8, HIDDEN_NF), jnp.float32),
    )(xh_atoms, xh_residues, t.reshape(1, 1),
      wa1, ba1, wa2, ba2, wr1, br1, wr2, br2, we, be)

    featpk = pl.pallas_call(
        _pack_kernel,
        in_specs=[
            pl.BlockSpec(memory_space=pltpu.SMEM),
            pl.BlockSpec((N_NODES + 8, HIDDEN_NF), lambda: (0, 0)),
        ],
        out_specs=pl.BlockSpec((N2, HIDDEN_NF), lambda: (0, 0)),
        out_shape=jax.ShapeDtypeStruct((N2, HIDDEN_NF), jnp.float32),
    )(jobs_pack, feat)

    gcl = p["gnn"]["gcl"]

    def stack(path):
        return jnp.stack([path(layer) for layer in gcl])

    def bdiag(w):
        return jnp.kron(jnp.eye(2, dtype=w.dtype), w)

    w1a_s = stack(lambda q: q["edge_mlp"]["l1"]["w"][:HIDDEN_NF, :])
    w1b_s = stack(lambda q: q["edge_mlp"]["l1"]["w"][HIDDEN_NF:, :])
    b1_s = stack(lambda q: q["edge_mlp"]["l1"]["b"].reshape(1, HIDDEN_NF))
    w2d_s = stack(lambda q: bdiag(q["edge_mlp"]["l2"]["w"]))
    b2d_s = stack(lambda q: jnp.tile(q["edge_mlp"]["l2"]["b"].reshape(1, HIDDEN_NF), (1, 2)))
    w3f_s = stack(lambda q: q["node_mlp"]["l1"]["w"][:HIDDEN_NF, :])
    w3a_s = stack(lambda q: q["node_mlp"]["l1"]["w"][HIDDEN_NF:, :])
    b3_s = stack(lambda q: q["node_mlp"]["l1"]["b"].reshape(1, HIDDEN_NF))
    w4_s = stack(lambda q: q["node_mlp"]["l2"]["w"])
    b4_s = stack(lambda q: q["node_mlp"]["l2"]["b"].reshape(1, HIDDEN_NF))

    bias2_s = b2d_s[:, None, :, :] + vbias[None, :, :, :]

    def wspec(shape):
        nd = len(shape) - 1
        return pl.BlockSpec((1,) + shape[1:], lambda l: (l,) + (0,) * nd)

    featpk = pl.pallas_call(
        _gcl_kernel,
        grid=(N_LAYERS,),
        in_specs=[
            pl.BlockSpec(memory_space=pltpu.SMEM),
            pl.BlockSpec((N2, HIDDEN_NF), lambda l: (0, 0)),
            wspec(bias2_s.shape),
            wspec(w1a_s.shape), wspec(b1_s.shape), wspec(w1b_s.shape),
            wspec(w2d_s.shape),
            wspec(w3f_s.shape), wspec(w3a_s.shape), wspec(b3_s.shape),
            wspec(w4_s.shape), wspec(b4_s.shape),
        ],
        out_specs=pl.BlockSpec((N2, HIDDEN_NF), lambda l: (0, 0)),
        out_shape=jax.ShapeDtypeStruct((N2, HIDDEN_NF), jnp.float32),
        scratch_shapes=[pltpu.VMEM((2, N2, HIDDEN_NF), jnp.float32),
                        pltpu.VMEM((N2, FOLD), jnp.float32),
                        pltpu.VMEM((NT2, TJH, FOLD), jnp.float32),
                        pltpu.VMEM((N2, HIDDEN_NF), jnp.float32)],
    )(dst.reshape(1, N_BATCH + 1), featpk, bias2_s,
      w1a_s, b1_s, w1b_s, w2d_s, w3f_s, w3a_s, b3_s, w4_s, b4_s)

    featu = pl.pallas_call(
        _unpack_kernel,
        in_specs=[
            pl.BlockSpec(memory_space=pltpu.SMEM),
            pl.BlockSpec((N2, HIDDEN_NF), lambda: (0, 0)),
        ],
        out_specs=pl.BlockSpec((N_NODES + 8, HIDDEN_NF), lambda: (0, 0)),
        out_shape=jax.ShapeDtypeStruct((N_NODES + 8, HIDDEN_NF), jnp.float32),
    )(jobs_unpack, featpk)[:N_NODES]

    weo, beo = p["gnn"]["embedding_out"]["w"], p["gnn"]["embedding_out"]["b"].reshape(1, -1)
    wad1, bad1 = lin("atom_decoder", "l1")
    wad2, bad2 = lin("atom_decoder", "l2")
    wrd1, brd1 = lin("residue_decoder", "l1")
    wrd2, brd2 = lin("residue_decoder", "l2")

    outa, outr = pl.pallas_call(
        _post_kernel,
        out_shape=(jax.ShapeDtypeStruct((N_ATOMS, N_DIMS + ATOM_NF), jnp.float32),
                   jax.ShapeDtypeStruct((N_RES, N_DIMS + RESIDUE_NF), jnp.float32)),
    )(featu, mask.reshape(N_NODES, 1),
      weo, beo, wad1, bad1, wad2, bad2, wrd1, brd1, wrd2, brd2)

    return outa, outr


def kernel(xh_atoms, xh_residues, xh_intersh, xh_intershp, t,
           mask_atoms, mask_residues, mask_intersh, mask_intershp, params):
    return _run(xh_atoms, xh_residues, t, mask_atoms, mask_residues, params)


# speedup vs baseline: 1.0771x; 1.0222x over previous
import jax
import jax.numpy as jnp
from jax.experimental import pallas as pl
from jax.experimental.pallas import tpu as pltpu

N_DIMS = 3
ATOM_NF = 16
RESIDUE_NF = 21
JOINT_NF = 16
HIDDEN_NF = 64
N_LAYERS = 4
NORM_FACTOR = 100.0
N_BATCH = 16
N_ATOMS = 320
N_RES = 1600
N_NODES = N_ATOMS + N_RES

TJ = 128
TI = 128
TJH = TJ // 2
FOLD = 2 * HIDDEN_NF
N2 = N_NODES + N_BATCH * (TJ - 1) + 32
N2 = ((N2 + TJ - 1) // TJ) * TJ
NT2 = N2 // TJ


def _silu(x):
    return 0.5 * x * (1.0 + jnp.tanh(0.5 * x))


def _silu_fast(x):
    t = 0.5 * x
    th = jnp.tanh(t.astype(jnp.bfloat16)).astype(jnp.float32)
    return t + t * th


def _pre_kernel(xa_ref, xr_ref, t_ref,
                wa1, ba1, wa2, ba2,
                wr1, br1, wr2, br2,
                we, be,
                out_ref):
    xa = xa_ref[...]
    xr = xr_ref[...]
    ha = xa[:, N_DIMS:]
    hr = xr[:, N_DIMS:]
    ha = _silu(ha @ wa1[...] + ba1[...]) @ wa2[...] + ba2[...]
    hr = _silu(hr @ wr1[...] + br1[...]) @ wr2[...] + br2[...]
    we_full = we[...]
    we_x = we_full[:N_DIMS, :]
    we_h = we_full[N_DIMS:N_DIMS + JOINT_NF, :]
    we_t = we_full[N_DIMS + JOINT_NF:, :]
    tval = t_ref[...]
    tcontrib = tval * we_t
    fa = xa[:, :N_DIMS] @ we_x + ha @ we_h + tcontrib + be[...]
    fr = xr[:, :N_DIMS] @ we_x + hr @ we_h + tcontrib + be[...]
    out_ref[:N_ATOMS, :] = fa
    out_ref[N_ATOMS:N_NODES, :] = fr
    out_ref[N_NODES:, :] = jnp.zeros((8, HIDDEN_NF), jnp.float32)


def _copy_jobs(jobs_ref, src_full, dst_ref, n_jobs):
    def job(b, _):
        src = jobs_ref[0, b]
        dstt = jobs_ref[1, b]
        ln = jobs_ref[2, b]

        def chunk(k, __):
            st = jnp.maximum(0, jnp.minimum(k * 8, ln - 8))
            dst_ref[pl.ds(dstt + st, 8), :] = src_full[pl.ds(src + st, 8), :]
            return 0

        return jax.lax.fori_loop(0, (ln + 7) // 8, chunk, 0)

    jax.lax.fori_loop(0, n_jobs, job, 0)


def _pack_kernel(jobs_ref, feat_ref, out_ref):
    out_ref[...] = jnp.zeros((N2, HIDDEN_NF), jnp.float32)
    _copy_jobs(jobs_ref, feat_ref, out_ref, 2 * N_BATCH)


def _unpack_kernel(jobs_ref, featpk_ref, out_ref):
    out_ref[...] = jnp.zeros((N_NODES + 8, HIDDEN_NF), jnp.float32)
    _copy_jobs(jobs_ref, featpk_ref, out_ref, 2 * N_BATCH)


def _gcl_kernel(dst_ref,
                feat0_ref,
                bias2_ref,
                w1a, b1, w1b, w2d,
                w3f, w3a, b3, w4, b4,
                out_ref,
                fbuf,
                a2buf,
                bfbuf,
                aggbuf):
    l = pl.program_id(0)
    cur = jax.lax.rem(l, 2)
    nxt = 1 - cur

    @pl.when(l == 0)
    def _copy_in():
        fbuf[0] = feat0_ref[...]

    f = fbuf[cur]
    a = f @ w1a[0] + b1[0]
    a2buf[...] = jnp.concatenate([a, a], axis=1)
    b = f @ w1b[0]
    br = b.reshape(NT2, 2, TJH, HIDDEN_NF)
    bfbuf[...] = jnp.concatenate([br[:, 0], br[:, 1]], axis=-1)
    aggbuf[...] = jnp.zeros((N2, HIDDEN_NF), jnp.float32)

    def batch_loop(bi, _):
        c0 = dst_ref[0, bi] // TJ
        c1 = dst_ref[0, bi + 1] // TJ
        r0 = dst_ref[0, bi] // TI
        r1 = dst_ref[0, bi + 1] // TI

        def row(rt, __):
            ioff = rt * TI
            a2 = a2buf[pl.ds(ioff, TI), :]

            def col(j, acc):
                bf = bfbuf[j]
                u = _silu(a2[:, None, :] + bf[None, :, :])
                mp = u.reshape(TI * TJH, FOLD) @ w2d[0]
                m = _silu(mp.reshape(TI, TJH, FOLD) + bias2_ref[0, j])
                s = jnp.sum(m, axis=1)
                return acc + s[:, :HIDDEN_NF] + s[:, HIDDEN_NF:]

            acc = jax.lax.fori_loop(c0, c1, col,
                                    jnp.zeros((TI, HIDDEN_NF), jnp.float32))
            aggbuf[pl.ds(ioff, TI), :] = acc
            return 0

        jax.lax.fori_loop(r0, r1, row, 0)
        return 0

    jax.lax.fori_loop(0, N_BATCH, batch_loop, 0)

    agg = aggbuf[...] * (1.0 / NORM_FACTOR)
    tmp = _silu(f @ w3f[0] + agg @ w3a[0] + b3[0])
    newf = f + tmp @ w4[0] + b4[0]
    fbuf[nxt] = newf

    @pl.when(l == N_LAYERS - 1)
    def _emit():
        out_ref[...] = newf


def _post_kernel(feat_ref, maski_ref,
                 weo, beo,
                 wad1, bad1, wad2, bad2,
                 wrd1, brd1, wrd2, brd2,
                 outa_ref, outr_ref):
    feat = feat_ref[...]
    out = feat @ weo[...] + beo[...]
    vel = out[:, :N_DIMS]
    hfin = out[:, N_DIMS:N_DIMS + JOINT_NF]
    ha = _silu(hfin[:N_ATOMS] @ wad1[...] + bad1[...]) @ wad2[...] + bad2[...]
    hr = _silu(hfin[N_ATOMS:] @ wrd1[...] + brd1[...]) @ wrd2[...] + brd2[...]
    maski = maski_ref[...]
    batches = jax.lax.broadcasted_iota(jnp.int32, (N_NODES, N_BATCH), 1)
    onehot = (maski == batches).astype(jnp.float32)
    seg = jax.lax.dot_general(onehot, vel, (((0,), (0,)), ((), ())))
    cnt = jnp.sum(onehot, axis=0, keepdims=True).T
    mean = seg / jnp.maximum(cnt, 1.0)
    vel = vel - onehot @ mean
    outa_ref[...] = jnp.concatenate([vel[:N_ATOMS], ha], axis=1)
    outr_ref[...] = jnp.concatenate([vel[N_ATOMS:], hr], axis=1)


@jax.jit
def _run(xh_atoms, xh_residues, t, mask_atoms, mask_residues, params):
    mask = jnp.concatenate([mask_atoms, mask_residues]).astype(jnp.int32)
    batch_ids = jnp.arange(N_BATCH + 1, dtype=jnp.int32)
    apos = jnp.searchsorted(mask_atoms.astype(jnp.int32), batch_ids).astype(jnp.int32)
    rpos = jnp.searchsorted(mask_residues.astype(jnp.int32), batch_ids).astype(jnp.int32)
    ca = apos[1:] - apos[:-1]
    cr = rpos[1:] - rpos[:-1]

    cnt = ca + cr
    padc = ((cnt + TJ - 1) // TJ) * TJ
    dst = jnp.concatenate([jnp.zeros((1,), jnp.int32),
                           jnp.cumsum(padc).astype(jnp.int32)])
    npad = dst[-1]

    rowidx = jnp.arange(N2, dtype=jnp.int32)
    brow = jnp.clip(jnp.searchsorted(dst, rowidx, side="right") - 1, 0, N_BATCH - 1)
    off = rowidx - dst[brow]
    valid = (off < cnt[brow]) & (rowidx < npad)

    src_a, dst_a = apos[:-1], dst[:-1]
    src_r, dst_r = N_ATOMS + rpos[:-1], dst[:-1] + ca
    jobs_pack = jnp.stack([
        jnp.stack([src_a, src_r], axis=1).reshape(-1),
        jnp.stack([dst_a, dst_r], axis=1).reshape(-1),
        jnp.stack([ca, cr], axis=1).reshape(-1),
    ]).astype(jnp.int32)
    jobs_unpack = jnp.stack([
        jnp.concatenate([dst_a, dst_r]),
        jnp.concatenate([src_a, src_r]),
        jnp.concatenate([ca, cr]),
    ]).astype(jnp.int32)

    vf = jnp.where(valid, 0.0, -1e9).astype(jnp.float32).reshape(NT2, 2, TJH)
    vbias = jnp.concatenate(
        [jnp.broadcast_to(vf[:, 0, :, None], (NT2, TJH, HIDDEN_NF)),
         jnp.broadcast_to(vf[:, 1, :, None], (NT2, TJH, HIDDEN_NF))], axis=2)

    p = params

    def lin(name, sub=None):
        q = p[name] if sub is None else p[name][sub]
        return q["w"], q["b"].reshape(1, -1)

    wa1, ba1 = lin("atom_encoder", "l1")
    wa2, ba2 = lin("atom_encoder", "l2")
    wr1, br1 = lin("residue_encoder", "l1")
    wr2, br2 = lin("residue_encoder", "l2")
    we, be = p["gnn"]["embedding"]["w"], p["gnn"]["embedding"]["b"].reshape(1, -1)

    feat = pl.pallas_call(
        _pre_kernel,
        out_shape=jax.ShapeDtypeStruct((N_NODES + 8, HIDDEN_NF), jnp.float32),
    )(xh_atoms, xh_residues, t.reshape(1, 1),
      wa1, ba1, wa2, ba2, wr1, br1, wr2, br2, we, be)

    featpk = pl.pallas_call(
        _pack_kernel,
        in_specs=[
            pl.BlockSpec(memory_space=pltpu.SMEM),
            pl.BlockSpec((N_NODES + 8, HIDDEN_NF), lambda: (0, 0)),
        ],
        out_specs=pl.BlockSpec((N2, HIDDEN_NF), lambda: (0, 0)),
        out_shape=jax.ShapeDtypeStruct((N2, HIDDEN_NF), jnp.float32),
    )(jobs_pack, feat)

    gcl = p["gnn"]["gcl"]

    def stack(path):
        return jnp.stack([path(layer) for layer in gcl])

    def bdiag(w):
        return jnp.kron(jnp.eye(2, dtype=w.dtype), w)

    w1a_s = stack(lambda q: q["edge_mlp"]["l1"]["w"][:HIDDEN_NF, :])
    w1b_s = stack(lambda q: q["edge_mlp"]["l1"]["w"][HIDDEN_NF:, :])
    b1_s = stack(lambda q: q["edge_mlp"]["l1"]["b"].reshape(1, HIDDEN_NF))
    w2d_s = stack(lambda q: bdiag(q["edge_mlp"]["l2"]["w"]))
    b2d_s = stack(lambda q: jnp.tile(q["edge_mlp"]["l2"]["b"].reshape(1, HIDDEN_NF), (1, 2)))
    w3f_s = stack(lambda q: q["node_mlp"]["l1"]["w"][:HIDDEN_NF, :])
    w3a_s = stack(lambda q: q["node_mlp"]["l1"]["w"][HIDDEN_NF:, :])
    b3_s = stack(lambda q: q["node_mlp"]["l1"]["b"].reshape(1, HIDDEN_NF))
    w4_s = stack(lambda q: q["node_mlp"]["l2"]["w"])
    b4_s = stack(lambda q: q["node_mlp"]["l2"]["b"].reshape(1, HIDDEN_NF))

    bias2_s = b2d_s[:, None, :, :] + vbias[None, :, :, :]

    def wspec(shape):
        nd = len(shape) - 1
        return pl.BlockSpec((1,) + shape[1:], lambda l: (l,) + (0,) * nd)

    featpk = pl.pallas_call(
        _gcl_kernel,
        grid=(N_LAYERS,),
        in_specs=[
            pl.BlockSpec(memory_space=pltpu.SMEM),
            pl.BlockSpec((N2, HIDDEN_NF), lambda l: (0, 0)),
            wspec(bias2_s.shape),
            wspec(w1a_s.shape), wspec(b1_s.shape), wspec(w1b_s.shape),
            wspec(w2d_s.shape),
            wspec(w3f_s.shape), wspec(w3a_s.shape), wspec(b3_s.shape),
            wspec(w4_s.shape), wspec(b4_s.shape),
        ],
        out_specs=pl.BlockSpec((N2, HIDDEN_NF), lambda l: (0, 0)),
        out_shape=jax.ShapeDtypeStruct((N2, HIDDEN_NF), jnp.float32),
        scratch_shapes=[pltpu.VMEM((2, N2, HIDDEN_NF), jnp.float32),
                        pltpu.VMEM((N2, FOLD), jnp.float32),
                        pltpu.VMEM((NT2, TJH, FOLD), jnp.float32),
                        pltpu.VMEM((N2, HIDDEN_NF), jnp.float32)],
    )(dst.reshape(1, N_BATCH + 1), featpk, bias2_s,
      w1a_s, b1_s, w1b_s, w2d_s, w3f_s, w3a_s, b3_s, w4_s, b4_s)

    featu = pl.pallas_call(
        _unpack_kernel,
        in_specs=[
            pl.BlockSpec(memory_space=pltpu.SMEM),
            pl.BlockSpec((N2, HIDDEN_NF), lambda: (0, 0)),
        ],
        out_specs=pl.BlockSpec((N_NODES + 8, HIDDEN_NF), lambda: (0, 0)),
        out_shape=jax.ShapeDtypeStruct((N_NODES + 8, HIDDEN_NF), jnp.float32),
    )(jobs_unpack, featpk)[:N_NODES]

    weo, beo = p["gnn"]["embedding_out"]["w"], p["gnn"]["embedding_out"]["b"].reshape(1, -1)
    wad1, bad1 = lin("atom_decoder", "l1")
    wad2, bad2 = lin("atom_decoder", "l2")
    wrd1, brd1 = lin("residue_decoder", "l1")
    wrd2, brd2 = lin("residue_decoder", "l2")

    outa, outr = pl.pallas_call(
        _post_kernel,
        out_shape=(jax.ShapeDtypeStruct((N_ATOMS, N_DIMS + ATOM_NF), jnp.float32),
                   jax.ShapeDtypeStruct((N_RES, N_DIMS + RESIDUE_NF), jnp.float32)),
    )(featu, mask.reshape(N_NODES, 1),
      weo, beo, wad1, bad1, wad2, bad2, wrd1, brd1, wrd2, brd2)

    return outa, outr


def kernel(xh_atoms, xh_residues, xh_intersh, xh_intershp, t,
           mask_atoms, mask_residues, mask_intersh, mask_intershp, params):
    return _run(xh_atoms, xh_residues, t, mask_atoms, mask_residues, params)
